# baseline (speedup 1.0000x reference)
import dataclasses
import functools

import jax
import jax.numpy as jnp
from jax import lax
from jax.experimental import pallas as pl
from jax.experimental.pallas import tpu as pltpu
from jax.experimental.pallas import tpu_sc as plsc

RAW = 40962
NEW = RAW * 4 - 6
M = NEW - RAW
BC = 128
NC, NS, L = 2, 16, 16
NW = NC * NS
NB = 4

W_T = 128
T_PW = 1296
T_PAD = NW * T_PW
T_FULL = T_PW // W_T
T_REM = T_PW % W_T

W_D = 64
D_PW = M // NW
D_CHUNKS = D_PW // W_D

_mesh = plsc.VectorSubcoreMesh(core_axis_name="c", subcore_axis_name="s")

_cp = pltpu.CompilerParams()
if "needs_layout_passes" in pltpu.CompilerParams.__dataclass_fields__:
    _cp = dataclasses.replace(_cp, needs_layout_passes=False)


@functools.partial(
    pl.kernel,
    mesh=_mesh,
    compiler_params=_cp,
    out_type=[
        jax.ShapeDtypeStruct((T_PAD, BC), jnp.float32),
        jax.ShapeDtypeStruct((M, BC), jnp.float32),
    ],
    scratch_types=[
        pltpu.VMEM((T_PW,), jnp.int32),
        pltpu.VMEM((2 * D_PW,), jnp.int32),
        pltpu.VMEM((2 * W_D, BC), jnp.float32),
        pltpu.VMEM((2 * W_D, BC), jnp.float32),
        pltpu.VMEM((2 * W_D, BC), jnp.float32),
        pltpu.VMEM((2 * W_D, BC), jnp.float32),
        pltpu.VMEM((W_D, BC), jnp.float32),
        pltpu.VMEM((W_D, BC), jnp.float32),
        pltpu.VMEM((W_D, BC), jnp.float32),
        pltpu.VMEM((W_D, BC), jnp.float32),
        pltpu.SemaphoreType.DMA,
        pltpu.SemaphoreType.DMA,
        pltpu.SemaphoreType.DMA,
        pltpu.SemaphoreType.DMA,
        pltpu.SemaphoreType.DMA,
        pltpu.SemaphoreType.DMA,
        pltpu.SemaphoreType.DMA,
        pltpu.SemaphoreType.DMA,
    ],
)
def _sc_gather_mean(xt_hbm, top_hbm, down_hbm, out1_hbm, out2_hbm,
                    idx_t, idx_d, rows0, rows1, rows2, rows3,
                    o0, o1, o2, o3, g0, g1, g2, g3, s0, s1, s2, s3):
    wid = lax.axis_index("s") * NC + lax.axis_index("c")
    rows = (rows0, rows1, rows2, rows3)
    o = (o0, o1, o2, o3)
    gs = (g0, g1, g2, g3)
    ss = (s0, s1, s2, s3)
    tbase = wid * T_PW
    dbase = wid * D_PW

    pltpu.sync_copy(top_hbm.at[pl.ds(tbase, T_PW)], idx_t)
    pltpu.sync_copy(down_hbm.at[pl.ds(2 * dbase, 2 * D_PW)], idx_d)

    @pl.loop(0, T_PW // L)
    def _(k):
        sl = pl.ds(k * L, L)
        idx_t[sl] = idx_t[sl] // 7

    @pl.loop(0, (2 * D_PW) // L)
    def _(k):
        sl = pl.ds(k * L, L)
        idx_d[sl] = idx_d[sl] // 7

    def tg(c, b):
        pltpu.async_copy(xt_hbm.at[idx_t.at[pl.ds(c * W_T, W_T)]],
                         rows[b], gs[b])

    def twait(b):
        pltpu.make_async_copy(xt_hbm.at[idx_t.at[pl.ds(0, W_T)]],
                              rows[b], gs[b]).wait()

    def tstore(c, b):
        pltpu.sync_copy(rows[b], out1_hbm.at[pl.ds(tbase + c * W_T, W_T)])

    for b in range(NB):
        tg(b, b)
    for c in range(T_FULL):
        b = c % NB
        twait(b)
        tstore(c, b)
        if c + NB < T_FULL:
            tg(c + NB, b)
    pltpu.sync_copy(xt_hbm.at[idx_t.at[pl.ds(T_FULL * W_T, T_REM)]],
                    rows0.at[pl.ds(0, T_REM)])
    pltpu.sync_copy(rows0.at[pl.ds(0, T_REM)],
                    out1_hbm.at[pl.ds(tbase + T_FULL * W_T, T_REM)])

    def dg(c, b):
        pltpu.async_copy(xt_hbm.at[idx_d.at[pl.ds(c * 2 * W_D, 2 * W_D)]],
                         rows[b], gs[b])

    def dwait(b):
        pltpu.make_async_copy(xt_hbm.at[idx_d.at[pl.ds(0, 2 * W_D)]],
                              rows[b], gs[b]).wait()

    def dcompute(b, ob):
        @pl.loop(0, W_D, step=4)
        def _(i0):
            for u in range(4):
                i = i0 + u
                for k in range(BC // L):
                    sl = pl.ds(k * L, L)
                    o[ob][i, sl] = \
                        (rows[b][2 * i, sl] + rows[b][2 * i + 1, sl]) * 0.5

    def dstore(c, ob):
        pltpu.async_copy(o[ob], out2_hbm.at[pl.ds(dbase + c * W_D, W_D)],
                         ss[ob])

    def dswait(ob):
        pltpu.make_async_copy(o[ob], out2_hbm.at[pl.ds(0, W_D)],
                              ss[ob]).wait()

    for b in range(NB):
        dg(b, b)
    for c in range(NB):
        b = c % NB
        dwait(b)
        dcompute(b, b)
        dg(c + NB, b)
        dstore(c, b)

    @pl.loop(NB, D_CHUNKS - NB, step=NB)
    def _(ci):
        for b in range(NB):
            c = ci + b
            dwait(b)
            dswait(b)
            dcompute(b, b)
            dg(c + NB, b)
            dstore(c, b)

    for boff in range(NB):
        c = D_CHUNKS - NB + boff
        b = c % NB
        dwait(b)
        dswait(b)
        dcompute(b, b)
        dstore(c, b)
    for b in range(NB):
        dswait(b)


def kernel(x, upconv_top_index, upconv_down_index):
    B, C, R = x.shape
    xt = x.reshape(B * C, R).T
    pad_vals = (jnp.arange(T_PAD - R, dtype=jnp.int32) % R) * 7
    top_full = jnp.concatenate([upconv_top_index, pad_vals])
    out1, out2 = _sc_gather_mean(xt, top_full, upconv_down_index)
    o1 = out1[:R].T.reshape(B, C, R)
    o2 = out2.T.reshape(B, C, M)
    return jnp.concatenate([o1, o2], axis=2)


# speedup vs baseline: 1.4497x; 1.4497x over previous
import dataclasses
import functools

import jax
import jax.numpy as jnp
from jax import lax
from jax.experimental import pallas as pl
from jax.experimental.pallas import tpu as pltpu
from jax.experimental.pallas import tpu_sc as plsc

RAW = 40962
NEW = RAW * 4 - 6
M = NEW - RAW
BC = 128
NC, NS, L = 2, 16, 16
NW = NC * NS
NB = 4

W_T = 128
T_PW = 1296
T_PAD = NW * T_PW
T_FULL = T_PW // W_T
T_REM = T_PW % W_T

W_D = 64
D_PW = M // NW
D_CHUNKS = D_PW // W_D

_mesh = plsc.VectorSubcoreMesh(core_axis_name="c", subcore_axis_name="s")

_cp = pltpu.CompilerParams()
if "needs_layout_passes" in pltpu.CompilerParams.__dataclass_fields__:
    _cp = dataclasses.replace(_cp, needs_layout_passes=False)


@functools.partial(
    pl.kernel,
    mesh=_mesh,
    compiler_params=_cp,
    out_type=[
        jax.ShapeDtypeStruct((T_PAD, BC), jnp.float32),
        jax.ShapeDtypeStruct((M, BC), jnp.float32),
    ],
    scratch_types=[
        pltpu.VMEM((T_PW,), jnp.int32),
        pltpu.VMEM((2 * D_PW,), jnp.int32),
        pltpu.VMEM((2 * W_D, BC), jnp.float32),
        pltpu.VMEM((2 * W_D, BC), jnp.float32),
        pltpu.VMEM((2 * W_D, BC), jnp.float32),
        pltpu.VMEM((2 * W_D, BC), jnp.float32),
        pltpu.VMEM((W_D, BC), jnp.float32),
        pltpu.VMEM((W_D, BC), jnp.float32),
        pltpu.VMEM((W_D, BC), jnp.float32),
        pltpu.VMEM((W_D, BC), jnp.float32),
        pltpu.SemaphoreType.DMA,
        pltpu.SemaphoreType.DMA,
        pltpu.SemaphoreType.DMA,
        pltpu.SemaphoreType.DMA,
        pltpu.SemaphoreType.DMA,
        pltpu.SemaphoreType.DMA,
        pltpu.SemaphoreType.DMA,
        pltpu.SemaphoreType.DMA,
    ],
)
def _sc_gather_mean(xt_hbm, top_hbm, down_hbm, out1_hbm, out2_hbm,
                    idx_t, idx_d, rows0, rows1, rows2, rows3,
                    o0, o1, o2, o3, g0, g1, g2, g3, s0, s1, s2, s3):
    wid = lax.axis_index("s") * NC + lax.axis_index("c")
    rows = (rows0, rows1, rows2, rows3)
    o = (o0, o1, o2, o3)
    gs = (g0, g1, g2, g3)
    ss = (s0, s1, s2, s3)
    tbase = wid * T_PW
    dbase = wid * D_PW

    pltpu.sync_copy(top_hbm.at[pl.ds(tbase, T_PW)], idx_t)
    pltpu.sync_copy(down_hbm.at[pl.ds(2 * dbase, 2 * D_PW)], idx_d)

    @pl.loop(0, T_PW // L)
    def _(k):
        sl = pl.ds(k * L, L)
        idx_t[sl] = idx_t[sl] // 7

    @pl.loop(0, (2 * D_PW) // L)
    def _(k):
        sl = pl.ds(k * L, L)
        idx_d[sl] = idx_d[sl] // 7

    def tg(c, b):
        pltpu.async_copy(xt_hbm.at[idx_t.at[pl.ds(c * W_T, W_T)]],
                         rows[b], gs[b])

    def twait(b):
        pltpu.make_async_copy(xt_hbm.at[idx_t.at[pl.ds(0, W_T)]],
                              rows[b], gs[b]).wait()

    def tstore(c, b):
        pltpu.sync_copy(rows[b], out1_hbm.at[pl.ds(tbase + c * W_T, W_T)])

    for b in range(NB):
        tg(b, b)
    for c in range(T_FULL):
        b = c % NB
        twait(b)
        tstore(c, b)
        if c + NB < T_FULL:
            tg(c + NB, b)
    pltpu.sync_copy(xt_hbm.at[idx_t.at[pl.ds(T_FULL * W_T, T_REM)]],
                    rows0.at[pl.ds(0, T_REM)])
    pltpu.sync_copy(rows0.at[pl.ds(0, T_REM)],
                    out1_hbm.at[pl.ds(tbase + T_FULL * W_T, T_REM)])

    def dg(c, b):
        pltpu.async_copy(xt_hbm.at[idx_d.at[pl.ds(c * 2 * W_D, 2 * W_D)]],
                         rows[b], gs[b])

    def dwait(b):
        pltpu.make_async_copy(xt_hbm.at[idx_d.at[pl.ds(0, 2 * W_D)]],
                              rows[b], gs[b]).wait()

    def dcompute(b, ob):
        @plsc.parallel_loop(0, W_D, step=1, unroll=4)
        def _(i):
            for k in range(BC // L):
                sl = pl.ds(k * L, L)
                o[ob][i, sl] = \
                    (rows[b][2 * i, sl] + rows[b][2 * i + 1, sl]) * 0.5

    def dstore(c, ob):
        pltpu.async_copy(o[ob], out2_hbm.at[pl.ds(dbase + c * W_D, W_D)],
                         ss[ob])

    def dswait(ob):
        pltpu.make_async_copy(o[ob], out2_hbm.at[pl.ds(0, W_D)],
                              ss[ob]).wait()

    for b in range(NB):
        dg(b, b)
    for c in range(NB):
        b = c % NB
        dwait(b)
        dcompute(b, b)
        dg(c + NB, b)
        dstore(c, b)

    @pl.loop(NB, D_CHUNKS - NB, step=NB)
    def _(ci):
        for b in range(NB):
            c = ci + b
            dwait(b)
            dswait(b)
            dcompute(b, b)
            dg(c + NB, b)
            dstore(c, b)

    for boff in range(NB):
        c = D_CHUNKS - NB + boff
        b = c % NB
        dwait(b)
        dswait(b)
        dcompute(b, b)
        dstore(c, b)
    for b in range(NB):
        dswait(b)


def kernel(x, upconv_top_index, upconv_down_index):
    B, C, R = x.shape
    xt = x.reshape(B * C, R).T
    pad_vals = (jnp.arange(T_PAD - R, dtype=jnp.int32) % R) * 7
    top_full = jnp.concatenate([upconv_top_index, pad_vals])
    out1, out2 = _sc_gather_mean(xt, top_full, upconv_down_index)
    o1 = out1[:R].T.reshape(B, C, R)
    o2 = out2.T.reshape(B, C, M)
    return jnp.concatenate([o1, o2], axis=2)
